# baseline (speedup 1.0000x reference)
import functools

import jax
import jax.numpy as jnp
from jax import lax
from jax.experimental import pallas as pl
from jax.experimental.pallas import tpu as pltpu
from jax.experimental.pallas import tpu_sc as plsc

_N = 256
_NC2 = _N * (_N - 1) // 2
_B = 1024
_RB = 64
_NBLK = _N // _RB
_NC = 2
_NS = 16
_NW = _NC * _NS
_BPW = _B // _NW


def _sj_vec(lanes, c):
    jv = 16 * c + lanes
    return lax.shift_right_logical(jv * (jv - 1), 1)


def _assemble_block(ibuf, obuf, lanes, q):
    r0 = _RB * q
    r1 = r0 + _RB
    ncq = _N // 16
    full_u = tuple(c for c in range(ncq) if 16 * c >= r1)
    if full_u:
        sjs = tuple(_sj_vec(lanes, c) for c in full_u)

        @plsc.parallel_loop(r0, r1, unroll=2)
        def pu_full(i, sjs=sjs, full_u=full_u, r0=r0):
            ob = (i - r0) * _N
            for c, sj in zip(full_u, sjs):
                obuf[pl.ds(ob + 16 * c, 16)] = plsc.load_gather(ibuf, [sj + i])
    for c in range(ncq):
        if r0 < 16 * c < r1:
            sj = _sj_vec(lanes, c)

            @plsc.parallel_loop(r0, 16 * c, unroll=8)
            def pu(i, sj=sj, c=c, r0=r0):
                g = plsc.load_gather(ibuf, [sj + i])
                obuf[pl.ds((i - r0) * _N + 16 * c, 16)] = g
    full_l = tuple(c for c in range(ncq) if 16 * c + 16 <= r0)
    if full_l:
        @plsc.parallel_loop(r0, r1, unroll=2,
                            carry=jnp.int32(r0 * (r0 - 1) // 2))
        def p1_full(i, s_i, full_l=full_l, r0=r0):
            ob = (i - r0) * _N
            for c in full_l:
                obuf[pl.ds(ob + 16 * c, 16)] = ibuf[pl.ds(s_i + 16 * c, 16)]
            return s_i + i
    for c in range(ncq):
        llo = 16 * c + 16
        if r0 < llo < r1:
            @plsc.parallel_loop(llo, r1, unroll=8,
                                carry=jnp.int32(llo * (llo - 1) // 2))
            def p1(i, s_i, c=c, r0=r0):
                obuf[pl.ds((i - r0) * _N + 16 * c, 16)] = (
                    ibuf[pl.ds(s_i + 16 * c, 16)])
                return s_i + i
    for rl in range(_RB // 16):
        r = (r0 // 16) + rl
        jv = 16 * r + lanes

        @plsc.parallel_loop(16 * r, 16 * r + 16, unroll=8)
        def p3(i, jv=jv, rl=rl, r=r):
            mx = jnp.maximum(jv, i)
            mn = jnp.minimum(jv, i)
            idx = lax.shift_right_logical(mx * (mx - 1), 1) + mn
            g = plsc.load_gather(ibuf, [idx])
            val = jnp.where(jv == i, jnp.float32(0.0), g)
            obuf[pl.ds((rl * 16 + i - 16 * r) * _N + 16 * r, 16)] = val


def _tri_body(in_hbm, out_hbm, ibuf_a, obuf_0, obuf_1, obuf_2,
              obuf_3, sem_ia, sem_o0, sem_o1, sem_o2, sem_o3):
    cid = lax.axis_index("c")
    sid = lax.axis_index("s")
    wid = sid * _NC + cid
    lanes = lax.iota(jnp.int32, 16)
    b0 = wid * _BPW

    def do_batch(b, ibuf, first, knext):
        bufs = (obuf_0, obuf_1, obuf_2, obuf_3)
        sems = (sem_o0, sem_o1, sem_o2, sem_o3)
        for q in range(_NBLK):
            buf, sem = bufs[q], sems[q]
            pending = pltpu.make_async_copy(buf, out_hbm.at[b, q], sem)
            if not first:
                pending.wait()
            _assemble_block(ibuf, buf, lanes, q)
            pltpu.make_async_copy(buf, out_hbm.at[b, q], sem).start()
            if q == _NBLK - 1:
                @pl.when(knext < _BPW)
                def _prefetch():
                    pltpu.make_async_copy(
                        in_hbm.at[b + 1], ibuf, sem_ia).start()

    pltpu.make_async_copy(in_hbm.at[b0], ibuf_a, sem_ia).start()

    def step(k, carry):
        b = b0 + k
        pltpu.make_async_copy(in_hbm.at[b], ibuf_a, sem_ia).wait()
        do_batch(b, ibuf_a, first=False, knext=k + 1)
        return carry

    pltpu.make_async_copy(in_hbm.at[b0], ibuf_a, sem_ia).wait()
    do_batch(b0, ibuf_a, first=True, knext=1)

    lax.fori_loop(1, _BPW, step, jnp.int32(0))
    b_last = b0 + _BPW - 1
    for q, (buf, sem) in enumerate(
            zip((obuf_0, obuf_1, obuf_2, obuf_3),
                (sem_o0, sem_o1, sem_o2, sem_o3))):
        pltpu.make_async_copy(buf, out_hbm.at[b_last, q], sem).wait()


@functools.lru_cache(maxsize=1)
def _build():
    return pl.kernel(
        _tri_body,
        out_type=jax.ShapeDtypeStruct((_B, _NBLK, _RB * _N), jnp.float32),
        mesh=plsc.VectorSubcoreMesh(core_axis_name="c", subcore_axis_name="s"),
        scratch_types=[
            pltpu.VMEM((_NC2,), jnp.float32),
            pltpu.VMEM((_RB * _N,), jnp.float32),
            pltpu.VMEM((_RB * _N,), jnp.float32),
            pltpu.VMEM((_RB * _N,), jnp.float32),
            pltpu.VMEM((_RB * _N,), jnp.float32),
            pltpu.SemaphoreType.DMA,
            pltpu.SemaphoreType.DMA,
            pltpu.SemaphoreType.DMA,
            pltpu.SemaphoreType.DMA,
            pltpu.SemaphoreType.DMA,
        ],
        compiler_params=pltpu.CompilerParams(needs_layout_passes=False),
    )


def kernel(decompFE):
    out = _build()(decompFE)
    return out.reshape(_B, _N, _N)


# speedup vs baseline: 1.1162x; 1.1162x over previous
import functools

import jax
import jax.numpy as jnp
from jax import lax
from jax.experimental import pallas as pl
from jax.experimental.pallas import tpu as pltpu
from jax.experimental.pallas import tpu_sc as plsc

_N = 256
_NC2 = _N * (_N - 1) // 2
_B = 1024
_RB = 64
_NBLK = _N // _RB
_NC = 2
_NS = 16
_NW = _NC * _NS
_BPW = _B // _NW


def _sj_vec(lanes, c):
    jv = 16 * c + lanes
    return lax.shift_right_logical(jv * (jv - 1), 1)


def _assemble_block(ibuf, obuf, lanes, q):
    r0 = _RB * q
    r1 = r0 + _RB
    ncq = _N // 16
    full_u = tuple(c for c in range(ncq) if 16 * c >= r1)
    if full_u:
        sjs = tuple(_sj_vec(lanes, c) for c in full_u)

        @plsc.parallel_loop(r0, r1, unroll=2)
        def pu_full(i, sjs=sjs, full_u=full_u, r0=r0):
            ob = (i - r0) * _N
            for c, sj in zip(full_u, sjs):
                obuf[pl.ds(ob + 16 * c, 16)] = plsc.load_gather(ibuf, [sj + i])
    for c in range(ncq):
        if r0 < 16 * c < r1:
            sj = _sj_vec(lanes, c)

            @plsc.parallel_loop(r0, 16 * c, unroll=8)
            def pu(i, sj=sj, c=c, r0=r0):
                g = plsc.load_gather(ibuf, [sj + i])
                obuf[pl.ds((i - r0) * _N + 16 * c, 16)] = g
    full_l = tuple(c for c in range(ncq) if 16 * c + 16 <= r0)
    if full_l:
        @plsc.parallel_loop(r0, r1, unroll=2,
                            carry=jnp.int32(r0 * (r0 - 1) // 2))
        def p1_full(i, s_i, full_l=full_l, r0=r0):
            ob = (i - r0) * _N
            for c in full_l:
                obuf[pl.ds(ob + 16 * c, 16)] = ibuf[pl.ds(s_i + 16 * c, 16)]
            return s_i + i
    for c in range(ncq):
        llo = 16 * c + 16
        if r0 < llo < r1:
            @plsc.parallel_loop(llo, r1, unroll=8,
                                carry=jnp.int32(llo * (llo - 1) // 2))
            def p1(i, s_i, c=c, r0=r0):
                obuf[pl.ds((i - r0) * _N + 16 * c, 16)] = (
                    ibuf[pl.ds(s_i + 16 * c, 16)])
                return s_i + i
    for rl in range(_RB // 16):
        r = (r0 // 16) + rl
        jv = 16 * r + lanes

        @plsc.parallel_loop(16 * r, 16 * r + 16, unroll=8)
        def p3(i, jv=jv, rl=rl, r=r):
            mx = jnp.maximum(jv, i)
            mn = jnp.minimum(jv, i)
            idx = lax.shift_right_logical(mx * (mx - 1), 1) + mn
            g = plsc.load_gather(ibuf, [idx])
            val = jnp.where(jv == i, jnp.float32(0.0), g)
            obuf[pl.ds((rl * 16 + i - 16 * r) * _N + 16 * r, 16)] = val


def _tri_body(in_hbm, out_hbm, ibuf_a, ibuf_b, obuf_a, obuf_b,
              sem_ia, sem_ib, sem_oa, sem_ob):
    cid = lax.axis_index("c")
    sid = lax.axis_index("s")
    wid = sid * _NC + cid
    lanes = lax.iota(jnp.int32, 16)
    b0 = wid * _BPW

    def do_batch(b, ibuf, first):
        bufs = (obuf_a, obuf_b)
        sems = (sem_oa, sem_ob)
        for q in range(_NBLK):
            buf, sem = bufs[q % 2], sems[q % 2]
            pending = pltpu.make_async_copy(buf, out_hbm.at[b, q], sem)
            if first and q < 2:
                pass
            elif first:
                pending.wait()
            else:
                @pl.when(jnp.logical_or(b > wid * _BPW, q >= 2))
                def _w():
                    pending.wait()
            _assemble_block(ibuf, buf, lanes, q)
            pltpu.make_async_copy(buf, out_hbm.at[b, q], sem).start()

    pltpu.make_async_copy(in_hbm.at[b0], ibuf_a, sem_ia).start()

    def step(k, carry):
        b = b0 + 2 * k
        pltpu.make_async_copy(in_hbm.at[b + 1], ibuf_b, sem_ib).start()
        pltpu.make_async_copy(in_hbm.at[b], ibuf_a, sem_ia).wait()
        do_batch(b, ibuf_a, first=False)

        @pl.when(k < _BPW // 2 - 1)
        def _prefetch():
            pltpu.make_async_copy(in_hbm.at[b + 2], ibuf_a, sem_ia).start()

        pltpu.make_async_copy(in_hbm.at[b + 1], ibuf_b, sem_ib).wait()
        do_batch(b + 1, ibuf_b, first=False)
        return carry

    lax.fori_loop(0, _BPW // 2, step, jnp.int32(0))
    b_last = b0 + _BPW - 1
    pltpu.make_async_copy(obuf_a, out_hbm.at[b_last, 2], sem_oa).wait()
    pltpu.make_async_copy(obuf_b, out_hbm.at[b_last, 3], sem_ob).wait()


@functools.lru_cache(maxsize=1)
def _build():
    return pl.kernel(
        _tri_body,
        out_type=jax.ShapeDtypeStruct((_B, _NBLK, _RB * _N), jnp.float32),
        mesh=plsc.VectorSubcoreMesh(core_axis_name="c", subcore_axis_name="s"),
        scratch_types=[
            pltpu.VMEM((_NC2,), jnp.float32),
            pltpu.VMEM((_NC2,), jnp.float32),
            pltpu.VMEM((_RB * _N,), jnp.float32),
            pltpu.VMEM((_RB * _N,), jnp.float32),
            pltpu.SemaphoreType.DMA,
            pltpu.SemaphoreType.DMA,
            pltpu.SemaphoreType.DMA,
            pltpu.SemaphoreType.DMA,
        ],
        compiler_params=pltpu.CompilerParams(needs_layout_passes=False),
    )


def kernel(decompFE):
    out = _build()(decompFE)
    return out.reshape(_B, _N, _N)
